# baseline (speedup 1.0000x reference)
import jax
import jax.numpy as jnp
from jax.experimental import pallas as pl
from jax.experimental.pallas import tpu as pltpu

K = 16
H = 512
W = 512
NB = 4
BR = H // NB


def _body(classes_smem, emb_ref, mask_ref, cls_vec_ref, out_ref, acc_ref):
    k = pl.program_id(0)
    b = pl.program_id(1)

    @pl.when(jnp.logical_and(k == 0, b == 0))
    def _init():
        acc_ref[...] = jnp.zeros((K, 3, 128), jnp.float32)

    e = emb_ref[0]
    m = jnp.where(mask_ref[0], 1.0, 0.0)
    em = e * m

    def fold(x):
        return x.sum(axis=0).reshape(4, 128).sum(axis=0)

    acc_ref[k, 0] += fold(em)
    acc_ref[k, 1] += fold(em * e)
    acc_ref[k, 2] += fold(m)

    @pl.when(jnp.logical_and(k == K - 1, b == NB - 1))
    def _finish():
        p = acc_ref[...]
        s = p[:, 0, :].sum(axis=-1, keepdims=True)
        s2 = p[:, 1, :].sum(axis=-1, keepdims=True)
        c = p[:, 2, :].sum(axis=-1, keepdims=True)
        safe = jnp.maximum(c, 1.0)
        means = jnp.where(c > 0, s / safe, 0.0)
        var = jnp.where(c > 0, s2 / safe - means * means, 0.0)
        row = jax.lax.broadcasted_iota(jnp.int32, (K, K), 0)
        col = jax.lax.broadcasted_iota(jnp.int32, (K, K), 1)
        eye = (row == col).astype(jnp.float32)
        mcol = jnp.broadcast_to(means, (K, K))
        mrow = (mcol * eye).sum(axis=0, keepdims=True)
        diff = mcol - mrow
        cls = cls_vec_ref[...].astype(jnp.float32)
        ccol = (jnp.broadcast_to(cls, (K, K)) * eye).sum(axis=-1, keepdims=True)
        same = (jnp.broadcast_to(ccol, (K, K)) == cls).astype(jnp.float32)
        triu = (col > row).astype(jnp.float32)
        inter = jnp.sum(jnp.maximum(1.0 - diff * diff, 0.0) * same * triu)
        reg = jnp.mean(means * means)
        intra = jnp.mean(var)
        out_ref[...] = jnp.reshape(inter + reg + intra, (1, 1))


def kernel(pred_emb, gt_objmask, gt_classes):
    cls = gt_classes.astype(jnp.int32)
    grid_spec = pltpu.PrefetchScalarGridSpec(
        num_scalar_prefetch=1,
        grid=(K, NB),
        in_specs=[
            pl.BlockSpec((1, BR, W), lambda k, b, classes: (classes[k], b, 0)),
            pl.BlockSpec((1, BR, W), lambda k, b, classes: (k, b, 0)),
            pl.BlockSpec((1, K), lambda k, b, classes: (0, 0)),
        ],
        out_specs=pl.BlockSpec((1, 1), lambda k, b, classes: (0, 0)),
        scratch_shapes=[pltpu.VMEM((K, 3, 128), jnp.float32)],
    )
    loss = pl.pallas_call(
        _body,
        grid_spec=grid_spec,
        out_shape=jax.ShapeDtypeStruct((1, 1), jnp.float32),
    )(cls, pred_emb, gt_objmask, cls[None, :])
    return loss.reshape(1)


# speedup vs baseline: 1.8013x; 1.8013x over previous
import jax
import jax.numpy as jnp
from jax.experimental import pallas as pl
from jax.experimental.pallas import tpu as pltpu

K = 16
H = 512
W = 512
NB = 1
BR = H // NB


def _body(classes_smem, emb_ref, mask_ref, cls_vec_ref, out_ref, acc_ref):
    k = pl.program_id(0)
    b = pl.program_id(1)

    @pl.when(jnp.logical_and(k == 0, b == 0))
    def _init():
        acc_ref[...] = jnp.zeros((K, 3, 128), jnp.float32)

    e = emb_ref[0]
    m = jnp.where(mask_ref[0], 1.0, 0.0)
    em = e * m

    def fold(x):
        return x.sum(axis=0).reshape(4, 128).sum(axis=0)

    acc_ref[k, 0] += fold(em)
    acc_ref[k, 1] += fold(em * e)
    acc_ref[k, 2] += fold(m)

    @pl.when(jnp.logical_and(k == K - 1, b == NB - 1))
    def _finish():
        p = acc_ref[...]
        s = p[:, 0, :].sum(axis=-1, keepdims=True)
        s2 = p[:, 1, :].sum(axis=-1, keepdims=True)
        c = p[:, 2, :].sum(axis=-1, keepdims=True)
        safe = jnp.maximum(c, 1.0)
        means = jnp.where(c > 0, s / safe, 0.0)
        var = jnp.where(c > 0, s2 / safe - means * means, 0.0)
        row = jax.lax.broadcasted_iota(jnp.int32, (K, K), 0)
        col = jax.lax.broadcasted_iota(jnp.int32, (K, K), 1)
        eye = (row == col).astype(jnp.float32)
        mcol = jnp.broadcast_to(means, (K, K))
        mrow = (mcol * eye).sum(axis=0, keepdims=True)
        diff = mcol - mrow
        cls = cls_vec_ref[...].astype(jnp.float32)
        ccol = (jnp.broadcast_to(cls, (K, K)) * eye).sum(axis=-1, keepdims=True)
        same = (jnp.broadcast_to(ccol, (K, K)) == cls).astype(jnp.float32)
        triu = (col > row).astype(jnp.float32)
        inter = jnp.sum(jnp.maximum(1.0 - diff * diff, 0.0) * same * triu)
        reg = jnp.mean(means * means)
        intra = jnp.mean(var)
        out_ref[...] = jnp.reshape(inter + reg + intra, (1, 1))


def kernel(pred_emb, gt_objmask, gt_classes):
    cls = gt_classes.astype(jnp.int32)
    grid_spec = pltpu.PrefetchScalarGridSpec(
        num_scalar_prefetch=1,
        grid=(K, NB),
        in_specs=[
            pl.BlockSpec((1, BR, W), lambda k, b, classes: (classes[k], b, 0)),
            pl.BlockSpec((1, BR, W), lambda k, b, classes: (k, b, 0)),
            pl.BlockSpec((1, K), lambda k, b, classes: (0, 0)),
        ],
        out_specs=pl.BlockSpec((1, 1), lambda k, b, classes: (0, 0)),
        scratch_shapes=[pltpu.VMEM((K, 3, 128), jnp.float32)],
    )
    loss = pl.pallas_call(
        _body,
        grid_spec=grid_spec,
        out_shape=jax.ShapeDtypeStruct((1, 1), jnp.float32),
    )(cls, pred_emb, gt_objmask, cls[None, :])
    return loss.reshape(1)
